# baseline (speedup 1.0000x reference)
import functools

import jax
import jax.numpy as jnp
from jax import lax
from jax.experimental import pallas as pl
from jax.experimental.pallas import tpu as pltpu
from jax.experimental.pallas import tpu_sc as plsc

N = 10000
D = 128
T = 16
N_PAD = 10240
E_PER_TILE = 10240
B = 128
NCHUNK = E_PER_TILE // B
W = 8
NW = NCHUNK // W
ROWS_PER_SUBCORE = N_PAD // 16
DUMMY_ROW = N
BLK = 512
GRID = N_PAD // BLK
EPS = 1e-5


@functools.lru_cache(maxsize=None)
def _sc_agg(with_deg):
  mesh = plsc.VectorSubcoreMesh(core_axis_name="c", subcore_axis_name="s")
  d = D
  out_type = [jax.ShapeDtypeStruct((2, N_PAD, d), jnp.float32)]
  scratch = [
      pltpu.VMEM((2, W, B), jnp.int32),
      pltpu.VMEM((2, W, B), jnp.int32),
      pltpu.VMEM((2 * B, d), jnp.float32),
      pltpu.VMEM_SHARED((N_PAD, d), jnp.float32),
      pltpu.SemaphoreType.DMA,
      pltpu.SemaphoreType.DMA,
      pltpu.SemaphoreType.DMA,
      pltpu.SemaphoreType.DMA,
      pltpu.SemaphoreType.DMA,
  ]
  if with_deg:
    out_type.append(jax.ShapeDtypeStruct((32, N_PAD), jnp.float32))
    scratch.append(pltpu.VMEM((N_PAD,), jnp.float32))

  @functools.partial(
      pl.kernel, out_type=out_type, mesh=mesh, scratch_types=scratch,
      compiler_params=pltpu.CompilerParams(needs_layout_passes=False))
  def k(x_hbm, src_hbm, dst_hbm, out_hbm, *rest):
    if with_deg:
      (deg_hbm, srcw, dstw, rows_v, acc, sem_a, sem_b, sem_sa, sem_sb,
       sem_i, hist_v) = rest
    else:
      srcw, dstw, rows_v, acc, sem_a, sem_b, sem_sa, sem_sb, sem_i = rest
    c = lax.axis_index("c")
    s = lax.axis_index("s")
    wid = c * 16 + s

    zero16 = jnp.zeros((16,), jnp.float32)

    def zrow(r, carry):
      for z in range(d // 16):
        rows_v[r, pl.ds(z * 16, 16)] = zero16
      return carry
    lax.fori_loop(0, 2 * B, zrow, 0)
    r0 = s * ROWS_PER_SUBCORE
    for t in range(ROWS_PER_SUBCORE // (2 * B)):
      pltpu.sync_copy(rows_v, acc.at[pl.ds(r0 + t * 2 * B, 2 * B)])
    rem = ROWS_PER_SUBCORE % (2 * B)
    if rem:
      pltpu.sync_copy(
          rows_v.at[pl.ds(0, rem)],
          acc.at[pl.ds(r0 + ROWS_PER_SUBCORE - rem, rem)])
    if with_deg:
      def zhist(i, carry):
        hist_v[pl.ds(i * 16, 16)] = zero16
        return carry
      lax.fori_loop(0, N_PAD // 16, zhist, 0)
    plsc.subcore_barrier()

    ones16 = jnp.ones((16,), jnp.float32)
    bufs = (rows_v.at[pl.ds(0, B)], rows_v.at[pl.ds(B, B)])
    gsem = (sem_a, sem_b)
    ssem = (sem_sa, sem_sb)

    def load_window(w_start, slot):
      pltpu.async_copy(src_hbm.at[wid, pl.ds(w_start, W)], srcw.at[slot],
                       sem_i)
      pltpu.async_copy(dst_hbm.at[wid, pl.ds(w_start, W)], dstw.at[slot],
                       sem_i)

    def wait_window(slot):
      pltpu.make_async_copy(src_hbm.at[0, pl.ds(0, W)], srcw.at[slot],
                            sem_i).wait()
      pltpu.make_async_copy(dst_hbm.at[0, pl.ds(0, W)], dstw.at[slot],
                            sem_i).wait()

    NSPLIT = 4
    SP = B // NSPLIT

    def issue_gather(slot, k, par):
      for h in range(NSPLIT):
        pltpu.async_copy(x_hbm.at[srcw.at[slot, k, pl.ds(h * SP, SP)]],
                         rows_v.at[pl.ds(par * B + h * SP, SP)], gsem[par])

    def wait_gather(par):
      for h in range(NSPLIT):
        pltpu.make_async_copy(x_hbm.at[pl.ds(0, SP)],
                              rows_v.at[pl.ds(par * B + h * SP, SP)],
                              gsem[par]).wait()

    def wait_scatter(par):
      pltpu.make_async_copy(x_hbm.at[pl.ds(0, B)], bufs[par], ssem[par]).wait()

    def process_window(slot, i, w, have_next):
      for k in range(W):
        par = k % 2
        if k < W - 1:
          if k == 0 and slot == 0:
            @pl.when(i > 0)
            def _():
              wait_scatter(1 - par)
          else:
            wait_scatter(1 - par)
          issue_gather(slot, k + 1, 1 - par)
        else:
          @pl.when(have_next)
          def _():
            wait_scatter(1 - par)
            wait_window(1 - slot)
            issue_gather(1 - slot, 0, 1 - par)
        if with_deg:
          for z in range(B // 16):
            idx = dstw[slot, k, pl.ds(z * 16, 16)]
            plsc.addupdate_scatter(hist_v, [idx], ones16)
        wait_gather(par)
        pltpu.async_copy(bufs[par], acc.at[dstw.at[slot, k]], ssem[par],
                         add=True)
        if k == 0:
          @pl.when(have_next)
          def _():
            load_window((w + 1) * W, 1 - slot)

    pltpu.sync_copy(src_hbm.at[wid, pl.ds(0, W)], srcw.at[0])
    pltpu.sync_copy(dst_hbm.at[wid, pl.ds(0, W)], dstw.at[0])
    issue_gather(0, 0, 0)

    def body(i, carry):
      process_window(0, i, 2 * i, i >= 0)
      process_window(1, i, 2 * i + 1, i < NW // 2 - 1)
      return carry
    lax.fori_loop(0, NW // 2, body, 0)
    wait_scatter(0)
    wait_scatter(1)
    if with_deg:
      pltpu.sync_copy(hist_v, deg_hbm.at[wid])
    plsc.subcore_barrier()

    pltpu.sync_copy(acc.at[pl.ds(r0, ROWS_PER_SUBCORE)],
                    out_hbm.at[c, pl.ds(r0, ROWS_PER_SUBCORE)])

  return k


def _stat_dots(ohb, h):
  dn = (((0,), (0,)), ((), ()))
  s_c = lax.dot_general(ohb, h, dn, preferred_element_type=jnp.float32)
  q_c = lax.dot_general(ohb, h * h, dn, preferred_element_type=jnp.float32)
  return s_c, q_c


def _tc_layer1(part, dhist, w1, b1, oh):
  def body(part_ref, dh_ref, w_ref, b_ref, oh_ref, h_ref, degbc_ref, s_ref,
           q_ref, cnt_ref):
    g = pl.program_id(0)
    agg = part_ref[0] + part_ref[1]
    deg = lax.dot_general(dh_ref[...], jnp.ones((32, D), jnp.float32),
                          (((0,), (0,)), ((), ())),
                          preferred_element_type=jnp.float32)
    deg = jnp.maximum(deg, 1.0)
    t = agg / deg
    h = jnp.dot(t, w_ref[...], preferred_element_type=jnp.float32) + b_ref[0]
    h = jnp.maximum(h, 0.0)
    h_ref[...] = h
    degbc_ref[...] = deg
    ohb = oh_ref[...]
    s_c, q_c = _stat_dots(ohb, h)
    c_c = lax.dot_general(ohb, jnp.ones((BLK, D), jnp.float32),
                          (((0,), (0,)), ((), ())),
                          preferred_element_type=jnp.float32)

    @pl.when(g == 0)
    def _():
      s_ref[...] = s_c
      q_ref[...] = q_c
      cnt_ref[...] = c_c

    @pl.when(g > 0)
    def _():
      s_ref[...] += s_c
      q_ref[...] += q_c
      cnt_ref[...] += c_c

  return pl.pallas_call(
      body,
      grid=(GRID,),
      in_specs=[
          pl.BlockSpec((2, BLK, D), lambda g: (0, g, 0)),
          pl.BlockSpec((32, BLK), lambda g: (0, g)),
          pl.BlockSpec((D, D), lambda g: (0, 0)),
          pl.BlockSpec((1, D), lambda g: (0, 0)),
          pl.BlockSpec((BLK, T), lambda g: (g, 0)),
      ],
      out_specs=[
          pl.BlockSpec((BLK, D), lambda g: (g, 0)),
          pl.BlockSpec((BLK, D), lambda g: (g, 0)),
          pl.BlockSpec((T, D), lambda g: (0, 0)),
          pl.BlockSpec((T, D), lambda g: (0, 0)),
          pl.BlockSpec((T, D), lambda g: (0, 0)),
      ],
      out_shape=[
          jax.ShapeDtypeStruct((N_PAD, D), jnp.float32),
          jax.ShapeDtypeStruct((N_PAD, D), jnp.float32),
          jax.ShapeDtypeStruct((T, D), jnp.float32),
          jax.ShapeDtypeStruct((T, D), jnp.float32),
          jax.ShapeDtypeStruct((T, D), jnp.float32),
      ],
  )(part, dhist, w1, b1, oh)


def _tc_layer2(part, w2, b2, oh, degbc):
  def body(part_ref, w_ref, b_ref, oh_ref, deg_ref, h_ref, s_ref, q_ref):
    g = pl.program_id(0)
    agg = part_ref[0] + part_ref[1]
    t = agg / deg_ref[...]
    h = jnp.dot(t, w_ref[...], preferred_element_type=jnp.float32) + b_ref[0]
    h_ref[...] = h
    s_c, q_c = _stat_dots(oh_ref[...], h)

    @pl.when(g == 0)
    def _():
      s_ref[...] = s_c
      q_ref[...] = q_c

    @pl.when(g > 0)
    def _():
      s_ref[...] += s_c
      q_ref[...] += q_c

  return pl.pallas_call(
      body,
      grid=(GRID,),
      in_specs=[
          pl.BlockSpec((2, BLK, D), lambda g: (0, g, 0)),
          pl.BlockSpec((D, D), lambda g: (0, 0)),
          pl.BlockSpec((1, D), lambda g: (0, 0)),
          pl.BlockSpec((BLK, T), lambda g: (g, 0)),
          pl.BlockSpec((BLK, D), lambda g: (g, 0)),
      ],
      out_specs=[
          pl.BlockSpec((BLK, D), lambda g: (g, 0)),
          pl.BlockSpec((T, D), lambda g: (0, 0)),
          pl.BlockSpec((T, D), lambda g: (0, 0)),
      ],
      out_shape=[
          jax.ShapeDtypeStruct((N_PAD, D), jnp.float32),
          jax.ShapeDtypeStruct((T, D), jnp.float32),
          jax.ShapeDtypeStruct((T, D), jnp.float32),
      ],
  )(part, w2, b2, oh, degbc)


def _tc_norm(h, oh, s, q, cnt):
  def body(h_ref, oh_ref, s_ref, q_ref, cnt_ref, o_ref):
    c = jnp.maximum(cnt_ref[...], 1.0)
    mean = s_ref[...] / c
    var = jnp.maximum(q_ref[...] / c - mean * mean, 0.0)
    ohb = oh_ref[...]
    m_rows = jnp.dot(ohb, mean, preferred_element_type=jnp.float32)
    v_rows = jnp.dot(ohb, var, preferred_element_type=jnp.float32)
    o_ref[...] = (h_ref[...] - m_rows) * lax.rsqrt(v_rows + EPS)

  return pl.pallas_call(
      body,
      grid=(GRID,),
      in_specs=[
          pl.BlockSpec((BLK, D), lambda g: (g, 0)),
          pl.BlockSpec((BLK, T), lambda g: (g, 0)),
          pl.BlockSpec((T, D), lambda g: (0, 0)),
          pl.BlockSpec((T, D), lambda g: (0, 0)),
          pl.BlockSpec((T, D), lambda g: (0, 0)),
      ],
      out_specs=pl.BlockSpec((BLK, D), lambda g: (g, 0)),
      out_shape=jax.ShapeDtypeStruct((N, D), jnp.float32),
  )(h, oh, s, q, cnt)


@jax.jit
def kernel(features, edge_index, times, W1, b1, W2, b2):
  e = edge_index.shape[1]
  e_pad = 32 * E_PER_TILE
  src = edge_index[0].astype(jnp.int32)
  dst = edge_index[1].astype(jnp.int32)
  src3 = jnp.concatenate(
      [src, jnp.zeros((e_pad - e,), jnp.int32)]).reshape(32, NCHUNK, B)
  pad_dst = N + jnp.arange(e_pad - e, dtype=jnp.int32) % (N_PAD - N)
  dst3 = jnp.concatenate([dst, pad_dst]).reshape(32, NCHUNK, B)

  tpad = jnp.concatenate(
      [times.astype(jnp.int32), jnp.full((N_PAD - N,), 12, jnp.int32)])
  oh = (tpad[:, None] == jnp.arange(T, dtype=jnp.int32)[None, :]).astype(
      jnp.float32)

  b1r = b1.reshape(1, D)
  b2r = b2.reshape(1, D)

  part1, dhist = _sc_agg(True)(features, src3, dst3)
  h1, degbc, s1, q1, cnt = _tc_layer1(part1, dhist, W1, b1r, oh)
  h1n = _tc_norm(h1, oh, s1, q1, cnt)
  part2, = _sc_agg(False)(h1n, src3, dst3)
  h2, s2, q2 = _tc_layer2(part2, W2, b2r, oh, degbc)
  return _tc_norm(h2, oh, s2, q2, cnt)


# speedup vs baseline: 1.0150x; 1.0150x over previous
import functools

import jax
import jax.numpy as jnp
from jax import lax
from jax.experimental import pallas as pl
from jax.experimental.pallas import tpu as pltpu
from jax.experimental.pallas import tpu_sc as plsc

N = 10000
D = 128
T = 16
N_PAD = 10240
E_PER_TILE = 10240
B = 128
NCHUNK = E_PER_TILE // B
W = 8
NW = NCHUNK // W
ROWS_PER_SUBCORE = N_PAD // 16
DUMMY_ROW = N
BLK = 512
GRID = N_PAD // BLK
EPS = 1e-5


@functools.lru_cache(maxsize=None)
def _sc_agg(with_deg):
  mesh = plsc.VectorSubcoreMesh(core_axis_name="c", subcore_axis_name="s")
  d = D
  out_type = [jax.ShapeDtypeStruct((2, N_PAD, d), jnp.float32)]
  scratch = [
      pltpu.VMEM((2, W, B), jnp.int32),
      pltpu.VMEM((2, W, B), jnp.int32),
      pltpu.VMEM((2 * B, d), jnp.float32),
      pltpu.VMEM_SHARED((N_PAD, d), jnp.float32),
      pltpu.SemaphoreType.DMA,
      pltpu.SemaphoreType.DMA,
      pltpu.SemaphoreType.DMA,
      pltpu.SemaphoreType.DMA,
      pltpu.SemaphoreType.DMA,
  ]
  if with_deg:
    out_type.append(jax.ShapeDtypeStruct((32, N_PAD), jnp.float32))
    scratch.append(pltpu.VMEM((N_PAD,), jnp.float32))

  @functools.partial(
      pl.kernel, out_type=out_type, mesh=mesh, scratch_types=scratch,
      compiler_params=pltpu.CompilerParams(needs_layout_passes=False))
  def k(x_hbm, src_hbm, dst_hbm, out_hbm, *rest):
    if with_deg:
      (deg_hbm, srcw, dstw, rows_v, acc, sem_a, sem_b, sem_sa, sem_sb,
       sem_i, hist_v) = rest
    else:
      srcw, dstw, rows_v, acc, sem_a, sem_b, sem_sa, sem_sb, sem_i = rest
    c = lax.axis_index("c")
    s = lax.axis_index("s")
    wid = c * 16 + s

    zero16 = jnp.zeros((16,), jnp.float32)

    def zrow(r, carry):
      for z in range(d // 16):
        rows_v[r, pl.ds(z * 16, 16)] = zero16
      return carry
    lax.fori_loop(0, 2 * B, zrow, 0)
    r0 = s * ROWS_PER_SUBCORE
    for t in range(ROWS_PER_SUBCORE // (2 * B)):
      pltpu.sync_copy(rows_v, acc.at[pl.ds(r0 + t * 2 * B, 2 * B)])
    rem = ROWS_PER_SUBCORE % (2 * B)
    if rem:
      pltpu.sync_copy(
          rows_v.at[pl.ds(0, rem)],
          acc.at[pl.ds(r0 + ROWS_PER_SUBCORE - rem, rem)])
    if with_deg:
      def zhist(i, carry):
        hist_v[pl.ds(i * 16, 16)] = zero16
        return carry
      lax.fori_loop(0, N_PAD // 16, zhist, 0)
    plsc.subcore_barrier()

    ones16 = jnp.ones((16,), jnp.float32)
    bufs = (rows_v.at[pl.ds(0, B)], rows_v.at[pl.ds(B, B)])
    gsem = (sem_a, sem_b)
    ssem = (sem_sa, sem_sb)

    def load_window(w_start, slot):
      pltpu.async_copy(src_hbm.at[wid, pl.ds(w_start, W)], srcw.at[slot],
                       sem_i)
      pltpu.async_copy(dst_hbm.at[wid, pl.ds(w_start, W)], dstw.at[slot],
                       sem_i)

    def wait_window(slot):
      pltpu.make_async_copy(src_hbm.at[0, pl.ds(0, W)], srcw.at[slot],
                            sem_i).wait()
      pltpu.make_async_copy(dst_hbm.at[0, pl.ds(0, W)], dstw.at[slot],
                            sem_i).wait()

    HALVES = ((0, 64), (64, B - 64))

    def issue_gather(slot, k, par):
      for off, ln in HALVES:
        pltpu.async_copy(x_hbm.at[srcw.at[slot, k, pl.ds(off, ln)]],
                         rows_v.at[pl.ds(par * B + off, ln)], gsem[par])

    def wait_gather(par):
      for off, ln in HALVES:
        pltpu.make_async_copy(x_hbm.at[pl.ds(0, ln)],
                              rows_v.at[pl.ds(par * B + off, ln)],
                              gsem[par]).wait()

    def wait_scatter(par):
      pltpu.make_async_copy(x_hbm.at[pl.ds(0, B)], bufs[par], ssem[par]).wait()

    def process_window(slot, i, w, have_next):
      for k in range(W):
        par = k % 2
        if k < W - 1:
          if k == 0 and slot == 0:
            @pl.when(i > 0)
            def _():
              wait_scatter(1 - par)
          else:
            wait_scatter(1 - par)
          issue_gather(slot, k + 1, 1 - par)
        else:
          @pl.when(have_next)
          def _():
            wait_scatter(1 - par)
            wait_window(1 - slot)
            issue_gather(1 - slot, 0, 1 - par)
        if with_deg:
          for z in range(B // 16):
            idx = dstw[slot, k, pl.ds(z * 16, 16)]
            plsc.addupdate_scatter(hist_v, [idx], ones16)
          if B % 16:
            idx = dstw[slot, k, pl.ds(B - 16, 16)]
            tmask = lax.iota(jnp.int32, 16) >= (16 - B % 16)
            plsc.addupdate_scatter(hist_v, [idx], ones16, mask=tmask)
        wait_gather(par)
        pltpu.async_copy(bufs[par], acc.at[dstw.at[slot, k]], ssem[par],
                         add=True)
        if k == 0:
          @pl.when(have_next)
          def _():
            load_window((w + 1) * W, 1 - slot)

    pltpu.sync_copy(src_hbm.at[wid, pl.ds(0, W)], srcw.at[0])
    pltpu.sync_copy(dst_hbm.at[wid, pl.ds(0, W)], dstw.at[0])
    issue_gather(0, 0, 0)

    def body(i, carry):
      process_window(0, i, 2 * i, i >= 0)
      process_window(1, i, 2 * i + 1, i < NW // 2 - 1)
      return carry
    lax.fori_loop(0, NW // 2, body, 0)
    wait_scatter(0)
    wait_scatter(1)
    if with_deg:
      pltpu.sync_copy(hist_v, deg_hbm.at[wid])
    plsc.subcore_barrier()

    pltpu.sync_copy(acc.at[pl.ds(r0, ROWS_PER_SUBCORE)],
                    out_hbm.at[c, pl.ds(r0, ROWS_PER_SUBCORE)])

  return k


def _stat_dots(ohb, h):
  dn = (((0,), (0,)), ((), ()))
  s_c = lax.dot_general(ohb, h, dn, preferred_element_type=jnp.float32)
  q_c = lax.dot_general(ohb, h * h, dn, preferred_element_type=jnp.float32)
  return s_c, q_c


def _tc_layer1(part, dhist, w1, b1, oh):
  def body(part_ref, dh_ref, w_ref, b_ref, oh_ref, h_ref, degbc_ref, s_ref,
           q_ref, cnt_ref):
    g = pl.program_id(0)
    agg = part_ref[0] + part_ref[1]
    deg = lax.dot_general(dh_ref[...], jnp.ones((32, D), jnp.float32),
                          (((0,), (0,)), ((), ())),
                          preferred_element_type=jnp.float32)
    deg = jnp.maximum(deg, 1.0)
    t = agg / deg
    h = jnp.dot(t, w_ref[...], preferred_element_type=jnp.float32) + b_ref[0]
    h = jnp.maximum(h, 0.0)
    h_ref[...] = h
    degbc_ref[...] = deg
    ohb = oh_ref[...]
    s_c, q_c = _stat_dots(ohb, h)
    c_c = lax.dot_general(ohb, jnp.ones((BLK, D), jnp.float32),
                          (((0,), (0,)), ((), ())),
                          preferred_element_type=jnp.float32)

    @pl.when(g == 0)
    def _():
      s_ref[...] = s_c
      q_ref[...] = q_c
      cnt_ref[...] = c_c

    @pl.when(g > 0)
    def _():
      s_ref[...] += s_c
      q_ref[...] += q_c
      cnt_ref[...] += c_c

  return pl.pallas_call(
      body,
      grid=(GRID,),
      in_specs=[
          pl.BlockSpec((2, BLK, D), lambda g: (0, g, 0)),
          pl.BlockSpec((32, BLK), lambda g: (0, g)),
          pl.BlockSpec((D, D), lambda g: (0, 0)),
          pl.BlockSpec((1, D), lambda g: (0, 0)),
          pl.BlockSpec((BLK, T), lambda g: (g, 0)),
      ],
      out_specs=[
          pl.BlockSpec((BLK, D), lambda g: (g, 0)),
          pl.BlockSpec((BLK, D), lambda g: (g, 0)),
          pl.BlockSpec((T, D), lambda g: (0, 0)),
          pl.BlockSpec((T, D), lambda g: (0, 0)),
          pl.BlockSpec((T, D), lambda g: (0, 0)),
      ],
      out_shape=[
          jax.ShapeDtypeStruct((N_PAD, D), jnp.float32),
          jax.ShapeDtypeStruct((N_PAD, D), jnp.float32),
          jax.ShapeDtypeStruct((T, D), jnp.float32),
          jax.ShapeDtypeStruct((T, D), jnp.float32),
          jax.ShapeDtypeStruct((T, D), jnp.float32),
      ],
  )(part, dhist, w1, b1, oh)


def _tc_layer2(part, w2, b2, oh, degbc):
  def body(part_ref, w_ref, b_ref, oh_ref, deg_ref, h_ref, s_ref, q_ref):
    g = pl.program_id(0)
    agg = part_ref[0] + part_ref[1]
    t = agg / deg_ref[...]
    h = jnp.dot(t, w_ref[...], preferred_element_type=jnp.float32) + b_ref[0]
    h_ref[...] = h
    s_c, q_c = _stat_dots(oh_ref[...], h)

    @pl.when(g == 0)
    def _():
      s_ref[...] = s_c
      q_ref[...] = q_c

    @pl.when(g > 0)
    def _():
      s_ref[...] += s_c
      q_ref[...] += q_c

  return pl.pallas_call(
      body,
      grid=(GRID,),
      in_specs=[
          pl.BlockSpec((2, BLK, D), lambda g: (0, g, 0)),
          pl.BlockSpec((D, D), lambda g: (0, 0)),
          pl.BlockSpec((1, D), lambda g: (0, 0)),
          pl.BlockSpec((BLK, T), lambda g: (g, 0)),
          pl.BlockSpec((BLK, D), lambda g: (g, 0)),
      ],
      out_specs=[
          pl.BlockSpec((BLK, D), lambda g: (g, 0)),
          pl.BlockSpec((T, D), lambda g: (0, 0)),
          pl.BlockSpec((T, D), lambda g: (0, 0)),
      ],
      out_shape=[
          jax.ShapeDtypeStruct((N_PAD, D), jnp.float32),
          jax.ShapeDtypeStruct((T, D), jnp.float32),
          jax.ShapeDtypeStruct((T, D), jnp.float32),
      ],
  )(part, w2, b2, oh, degbc)


def _tc_norm(h, oh, s, q, cnt):
  def body(h_ref, oh_ref, s_ref, q_ref, cnt_ref, o_ref):
    c = jnp.maximum(cnt_ref[...], 1.0)
    mean = s_ref[...] / c
    var = jnp.maximum(q_ref[...] / c - mean * mean, 0.0)
    ohb = oh_ref[...]
    m_rows = jnp.dot(ohb, mean, preferred_element_type=jnp.float32)
    v_rows = jnp.dot(ohb, var, preferred_element_type=jnp.float32)
    o_ref[...] = (h_ref[...] - m_rows) * lax.rsqrt(v_rows + EPS)

  return pl.pallas_call(
      body,
      grid=(GRID,),
      in_specs=[
          pl.BlockSpec((BLK, D), lambda g: (g, 0)),
          pl.BlockSpec((BLK, T), lambda g: (g, 0)),
          pl.BlockSpec((T, D), lambda g: (0, 0)),
          pl.BlockSpec((T, D), lambda g: (0, 0)),
          pl.BlockSpec((T, D), lambda g: (0, 0)),
      ],
      out_specs=pl.BlockSpec((BLK, D), lambda g: (g, 0)),
      out_shape=jax.ShapeDtypeStruct((N, D), jnp.float32),
  )(h, oh, s, q, cnt)


@jax.jit
def kernel(features, edge_index, times, W1, b1, W2, b2):
  e = edge_index.shape[1]
  e_pad = 32 * E_PER_TILE
  src = edge_index[0].astype(jnp.int32)
  dst = edge_index[1].astype(jnp.int32)
  src3 = jnp.concatenate(
      [src, jnp.zeros((e_pad - e,), jnp.int32)]).reshape(32, NCHUNK, B)
  pad_dst = N + jnp.arange(e_pad - e, dtype=jnp.int32) % (N_PAD - N)
  dst3 = jnp.concatenate([dst, pad_dst]).reshape(32, NCHUNK, B)

  tpad = jnp.concatenate(
      [times.astype(jnp.int32), jnp.full((N_PAD - N,), 12, jnp.int32)])
  oh = (tpad[:, None] == jnp.arange(T, dtype=jnp.int32)[None, :]).astype(
      jnp.float32)

  b1r = b1.reshape(1, D)
  b2r = b2.reshape(1, D)

  part1, dhist = _sc_agg(True)(features, src3, dst3)
  h1, degbc, s1, q1, cnt = _tc_layer1(part1, dhist, W1, b1r, oh)
  h1n = _tc_norm(h1, oh, s1, q1, cnt)
  part2, = _sc_agg(False)(h1n, src3, dst3)
  h2, s2, q2 = _tc_layer2(part2, W2, b2r, oh, degbc)
  return _tc_norm(h2, oh, s2, q2, cnt)


# speedup vs baseline: 1.0230x; 1.0079x over previous
import functools

import jax
import jax.numpy as jnp
from jax import lax
from jax.experimental import pallas as pl
from jax.experimental.pallas import tpu as pltpu
from jax.experimental.pallas import tpu_sc as plsc

N = 10000
D = 128
T = 16
N_PAD = 10240
E_PER_TILE = 10240
B = 128
NCHUNK = E_PER_TILE // B
W = 8
NW = NCHUNK // W
ROWS_PER_SUBCORE = N_PAD // 16
DUMMY_ROW = N
BLK = 1024
GRID = N_PAD // BLK
EPS = 1e-5


@functools.lru_cache(maxsize=None)
def _sc_agg(with_deg):
  mesh = plsc.VectorSubcoreMesh(core_axis_name="c", subcore_axis_name="s")
  d = D
  out_type = [jax.ShapeDtypeStruct((2, N_PAD, d), jnp.float32)]
  scratch = [
      pltpu.VMEM((2, W, B), jnp.int32),
      pltpu.VMEM((2, W, B), jnp.int32),
      pltpu.VMEM((2 * B, d), jnp.float32),
      pltpu.VMEM_SHARED((N_PAD, d), jnp.float32),
      pltpu.SemaphoreType.DMA,
      pltpu.SemaphoreType.DMA,
      pltpu.SemaphoreType.DMA,
      pltpu.SemaphoreType.DMA,
      pltpu.SemaphoreType.DMA,
  ]
  if with_deg:
    out_type.append(jax.ShapeDtypeStruct((32, N_PAD), jnp.float32))
    scratch.append(pltpu.VMEM((N_PAD,), jnp.float32))

  @functools.partial(
      pl.kernel, out_type=out_type, mesh=mesh, scratch_types=scratch,
      compiler_params=pltpu.CompilerParams(needs_layout_passes=False))
  def k(x_hbm, src_hbm, dst_hbm, out_hbm, *rest):
    if with_deg:
      (deg_hbm, srcw, dstw, rows_v, acc, sem_a, sem_b, sem_sa, sem_sb,
       sem_i, hist_v) = rest
    else:
      srcw, dstw, rows_v, acc, sem_a, sem_b, sem_sa, sem_sb, sem_i = rest
    c = lax.axis_index("c")
    s = lax.axis_index("s")
    wid = c * 16 + s

    zero16 = jnp.zeros((16,), jnp.float32)

    def zrow(r, carry):
      for z in range(d // 16):
        rows_v[r, pl.ds(z * 16, 16)] = zero16
      return carry
    lax.fori_loop(0, 2 * B, zrow, 0)
    r0 = s * ROWS_PER_SUBCORE
    for t in range(ROWS_PER_SUBCORE // (2 * B)):
      pltpu.sync_copy(rows_v, acc.at[pl.ds(r0 + t * 2 * B, 2 * B)])
    rem = ROWS_PER_SUBCORE % (2 * B)
    if rem:
      pltpu.sync_copy(
          rows_v.at[pl.ds(0, rem)],
          acc.at[pl.ds(r0 + ROWS_PER_SUBCORE - rem, rem)])
    if with_deg:
      def zhist(i, carry):
        hist_v[pl.ds(i * 16, 16)] = zero16
        return carry
      lax.fori_loop(0, N_PAD // 16, zhist, 0)
    plsc.subcore_barrier()

    ones16 = jnp.ones((16,), jnp.float32)
    bufs = (rows_v.at[pl.ds(0, B)], rows_v.at[pl.ds(B, B)])
    gsem = (sem_a, sem_b)
    ssem = (sem_sa, sem_sb)

    def load_window(w_start, slot):
      pltpu.async_copy(src_hbm.at[wid, pl.ds(w_start, W)], srcw.at[slot],
                       sem_i)
      pltpu.async_copy(dst_hbm.at[wid, pl.ds(w_start, W)], dstw.at[slot],
                       sem_i)

    def wait_window(slot):
      pltpu.make_async_copy(src_hbm.at[0, pl.ds(0, W)], srcw.at[slot],
                            sem_i).wait()
      pltpu.make_async_copy(dst_hbm.at[0, pl.ds(0, W)], dstw.at[slot],
                            sem_i).wait()

    HALVES = ((0, 64), (64, B - 64))

    def issue_gather(slot, k, par):
      for off, ln in HALVES:
        pltpu.async_copy(x_hbm.at[srcw.at[slot, k, pl.ds(off, ln)]],
                         rows_v.at[pl.ds(par * B + off, ln)], gsem[par])

    def wait_gather(par):
      for off, ln in HALVES:
        pltpu.make_async_copy(x_hbm.at[pl.ds(0, ln)],
                              rows_v.at[pl.ds(par * B + off, ln)],
                              gsem[par]).wait()

    def wait_scatter(par):
      pltpu.make_async_copy(x_hbm.at[pl.ds(0, B)], bufs[par], ssem[par]).wait()

    def process_window(slot, i, w, have_next):
      for k in range(W):
        par = k % 2
        if k < W - 1:
          if k == 0 and slot == 0:
            @pl.when(i > 0)
            def _():
              wait_scatter(1 - par)
          else:
            wait_scatter(1 - par)
          issue_gather(slot, k + 1, 1 - par)
        else:
          @pl.when(have_next)
          def _():
            wait_scatter(1 - par)
            wait_window(1 - slot)
            issue_gather(1 - slot, 0, 1 - par)
        if with_deg:
          for z in range(B // 16):
            idx = dstw[slot, k, pl.ds(z * 16, 16)]
            plsc.addupdate_scatter(hist_v, [idx], ones16)
          if B % 16:
            idx = dstw[slot, k, pl.ds(B - 16, 16)]
            tmask = lax.iota(jnp.int32, 16) >= (16 - B % 16)
            plsc.addupdate_scatter(hist_v, [idx], ones16, mask=tmask)
        wait_gather(par)
        pltpu.async_copy(bufs[par], acc.at[dstw.at[slot, k]], ssem[par],
                         add=True)
        if k == 0:
          @pl.when(have_next)
          def _():
            load_window((w + 1) * W, 1 - slot)

    pltpu.sync_copy(src_hbm.at[wid, pl.ds(0, W)], srcw.at[0])
    pltpu.sync_copy(dst_hbm.at[wid, pl.ds(0, W)], dstw.at[0])
    issue_gather(0, 0, 0)

    def body(i, carry):
      process_window(0, i, 2 * i, i >= 0)
      process_window(1, i, 2 * i + 1, i < NW // 2 - 1)
      return carry
    lax.fori_loop(0, NW // 2, body, 0)
    wait_scatter(0)
    wait_scatter(1)
    if with_deg:
      pltpu.sync_copy(hist_v, deg_hbm.at[wid])
    plsc.subcore_barrier()

    pltpu.sync_copy(acc.at[pl.ds(r0, ROWS_PER_SUBCORE)],
                    out_hbm.at[c, pl.ds(r0, ROWS_PER_SUBCORE)])

  return k


def _stat_dots(ohb, h):
  dn = (((0,), (0,)), ((), ()))
  s_c = lax.dot_general(ohb, h, dn, preferred_element_type=jnp.float32)
  q_c = lax.dot_general(ohb, h * h, dn, preferred_element_type=jnp.float32)
  return s_c, q_c


def _tc_layer1(part, dhist, w1, b1, oh):
  def body(part_ref, dh_ref, w_ref, b_ref, oh_ref, h_ref, degbc_ref, s_ref,
           q_ref, cnt_ref):
    g = pl.program_id(0)
    agg = part_ref[0] + part_ref[1]
    deg = lax.dot_general(dh_ref[...], jnp.ones((32, D), jnp.float32),
                          (((0,), (0,)), ((), ())),
                          preferred_element_type=jnp.float32)
    deg = jnp.maximum(deg, 1.0)
    t = agg / deg
    h = jnp.dot(t, w_ref[...], preferred_element_type=jnp.float32) + b_ref[0]
    h = jnp.maximum(h, 0.0)
    h_ref[...] = h
    degbc_ref[...] = deg
    ohb = oh_ref[...]
    s_c, q_c = _stat_dots(ohb, h)
    c_c = lax.dot_general(ohb, jnp.ones((BLK, D), jnp.float32),
                          (((0,), (0,)), ((), ())),
                          preferred_element_type=jnp.float32)

    @pl.when(g == 0)
    def _():
      s_ref[...] = s_c
      q_ref[...] = q_c
      cnt_ref[...] = c_c

    @pl.when(g > 0)
    def _():
      s_ref[...] += s_c
      q_ref[...] += q_c
      cnt_ref[...] += c_c

  return pl.pallas_call(
      body,
      grid=(GRID,),
      in_specs=[
          pl.BlockSpec((2, BLK, D), lambda g: (0, g, 0)),
          pl.BlockSpec((32, BLK), lambda g: (0, g)),
          pl.BlockSpec((D, D), lambda g: (0, 0)),
          pl.BlockSpec((1, D), lambda g: (0, 0)),
          pl.BlockSpec((BLK, T), lambda g: (g, 0)),
      ],
      out_specs=[
          pl.BlockSpec((BLK, D), lambda g: (g, 0)),
          pl.BlockSpec((BLK, D), lambda g: (g, 0)),
          pl.BlockSpec((T, D), lambda g: (0, 0)),
          pl.BlockSpec((T, D), lambda g: (0, 0)),
          pl.BlockSpec((T, D), lambda g: (0, 0)),
      ],
      out_shape=[
          jax.ShapeDtypeStruct((N_PAD, D), jnp.float32),
          jax.ShapeDtypeStruct((N_PAD, D), jnp.float32),
          jax.ShapeDtypeStruct((T, D), jnp.float32),
          jax.ShapeDtypeStruct((T, D), jnp.float32),
          jax.ShapeDtypeStruct((T, D), jnp.float32),
      ],
  )(part, dhist, w1, b1, oh)


def _tc_layer2(part, w2, b2, oh, degbc):
  def body(part_ref, w_ref, b_ref, oh_ref, deg_ref, h_ref, s_ref, q_ref):
    g = pl.program_id(0)
    agg = part_ref[0] + part_ref[1]
    t = agg / deg_ref[...]
    h = jnp.dot(t, w_ref[...], preferred_element_type=jnp.float32) + b_ref[0]
    h_ref[...] = h
    s_c, q_c = _stat_dots(oh_ref[...], h)

    @pl.when(g == 0)
    def _():
      s_ref[...] = s_c
      q_ref[...] = q_c

    @pl.when(g > 0)
    def _():
      s_ref[...] += s_c
      q_ref[...] += q_c

  return pl.pallas_call(
      body,
      grid=(GRID,),
      in_specs=[
          pl.BlockSpec((2, BLK, D), lambda g: (0, g, 0)),
          pl.BlockSpec((D, D), lambda g: (0, 0)),
          pl.BlockSpec((1, D), lambda g: (0, 0)),
          pl.BlockSpec((BLK, T), lambda g: (g, 0)),
          pl.BlockSpec((BLK, D), lambda g: (g, 0)),
      ],
      out_specs=[
          pl.BlockSpec((BLK, D), lambda g: (g, 0)),
          pl.BlockSpec((T, D), lambda g: (0, 0)),
          pl.BlockSpec((T, D), lambda g: (0, 0)),
      ],
      out_shape=[
          jax.ShapeDtypeStruct((N_PAD, D), jnp.float32),
          jax.ShapeDtypeStruct((T, D), jnp.float32),
          jax.ShapeDtypeStruct((T, D), jnp.float32),
      ],
  )(part, w2, b2, oh, degbc)


def _tc_norm(h, oh, s, q, cnt):
  def body(h_ref, oh_ref, s_ref, q_ref, cnt_ref, o_ref):
    c = jnp.maximum(cnt_ref[...], 1.0)
    mean = s_ref[...] / c
    var = jnp.maximum(q_ref[...] / c - mean * mean, 0.0)
    ohb = oh_ref[...]
    m_rows = jnp.dot(ohb, mean, preferred_element_type=jnp.float32)
    v_rows = jnp.dot(ohb, var, preferred_element_type=jnp.float32)
    o_ref[...] = (h_ref[...] - m_rows) * lax.rsqrt(v_rows + EPS)

  return pl.pallas_call(
      body,
      grid=(GRID,),
      in_specs=[
          pl.BlockSpec((BLK, D), lambda g: (g, 0)),
          pl.BlockSpec((BLK, T), lambda g: (g, 0)),
          pl.BlockSpec((T, D), lambda g: (0, 0)),
          pl.BlockSpec((T, D), lambda g: (0, 0)),
          pl.BlockSpec((T, D), lambda g: (0, 0)),
      ],
      out_specs=pl.BlockSpec((BLK, D), lambda g: (g, 0)),
      out_shape=jax.ShapeDtypeStruct((N, D), jnp.float32),
  )(h, oh, s, q, cnt)


@jax.jit
def kernel(features, edge_index, times, W1, b1, W2, b2):
  e = edge_index.shape[1]
  e_pad = 32 * E_PER_TILE
  src = edge_index[0].astype(jnp.int32)
  dst = edge_index[1].astype(jnp.int32)
  src3 = jnp.concatenate(
      [src, jnp.zeros((e_pad - e,), jnp.int32)]).reshape(32, NCHUNK, B)
  pad_dst = N + jnp.arange(e_pad - e, dtype=jnp.int32) % (N_PAD - N)
  dst3 = jnp.concatenate([dst, pad_dst]).reshape(32, NCHUNK, B)

  tpad = jnp.concatenate(
      [times.astype(jnp.int32), jnp.full((N_PAD - N,), 12, jnp.int32)])
  oh = (tpad[:, None] == jnp.arange(T, dtype=jnp.int32)[None, :]).astype(
      jnp.float32)

  b1r = b1.reshape(1, D)
  b2r = b2.reshape(1, D)

  part1, dhist = _sc_agg(True)(features, src3, dst3)
  h1, degbc, s1, q1, cnt = _tc_layer1(part1, dhist, W1, b1r, oh)
  h1n = _tc_norm(h1, oh, s1, q1, cnt)
  part2, = _sc_agg(False)(h1n, src3, dst3)
  h2, s2, q2 = _tc_layer2(part2, W2, b2r, oh, degbc)
  return _tc_norm(h2, oh, s2, q2, cnt)
